# baseline (speedup 1.0000x reference)
import functools

import jax
import jax.numpy as jnp
from jax import lax
from jax.experimental import pallas as pl
from jax.experimental.pallas import tpu as pltpu
from jax.experimental.pallas import tpu_sc as plsc

_TIME_STEPS = 1000
_EMBED_DIM = 128
_BATCH = 16384

_NUM_CORES = 2
_NUM_SUBCORES = 16
_NUM_WORKERS = _NUM_CORES * _NUM_SUBCORES
_B_PER_W = _BATCH // _NUM_WORKERS
_CHUNK = 128
_NCHUNKS = _B_PER_W // _CHUNK


_HALF = _B_PER_W // 2


def _gather_kernel(table_hbm, idx_hbm, out_hbm, idx_v, rows_v, sem_a, sem_b):
    wid = lax.axis_index("s") * _NUM_CORES + lax.axis_index("c")
    base = wid * _B_PER_W
    pltpu.sync_copy(idx_hbm.at[pl.ds(base, _B_PER_W)], idx_v)
    g0 = pltpu.async_copy(table_hbm.at[idx_v.at[pl.ds(0, _HALF)]],
                          rows_v.at[pl.ds(0, _HALF)], sem_a)
    g1 = pltpu.async_copy(table_hbm.at[idx_v.at[pl.ds(_HALF, _HALF)]],
                          rows_v.at[pl.ds(_HALF, _HALF)], sem_b)
    g0.wait()
    w0 = pltpu.async_copy(rows_v.at[pl.ds(0, _HALF)],
                          out_hbm.at[pl.ds(base, _HALF)], sem_a)
    g1.wait()
    w1 = pltpu.async_copy(rows_v.at[pl.ds(_HALF, _HALF)],
                          out_hbm.at[pl.ds(base + _HALF, _HALF)], sem_b)
    w0.wait()
    w1.wait()


@jax.jit
def _gather(embeddings, t):
    mesh = plsc.VectorSubcoreMesh(core_axis_name="c", subcore_axis_name="s")
    return pl.kernel(
        _gather_kernel,
        mesh=mesh,
        out_type=jax.ShapeDtypeStruct((_BATCH, _EMBED_DIM), jnp.float32),
        scratch_types=[
            pltpu.VMEM((_B_PER_W,), jnp.int32),
            pltpu.VMEM((_B_PER_W, _EMBED_DIM), jnp.float32),
            pltpu.SemaphoreType.DMA,
            pltpu.SemaphoreType.DMA,
        ],
    )(embeddings, t)


def kernel(x, t, embeddings):
    return _gather(embeddings, t)


# speedup vs baseline: 1.0571x; 1.0571x over previous
import functools

import jax
import jax.numpy as jnp
from jax import lax
from jax.experimental import pallas as pl
from jax.experimental.pallas import tpu as pltpu
from jax.experimental.pallas import tpu_sc as plsc

_TIME_STEPS = 1000
_EMBED_DIM = 128
_BATCH = 16384

_NUM_CORES = 2
_NUM_SUBCORES = 16
_NUM_WORKERS = _NUM_CORES * _NUM_SUBCORES
_B_PER_W = _BATCH // _NUM_WORKERS
_CHUNK = 128
_NCHUNKS = _B_PER_W // _CHUNK


def _gather_kernel(table_hbm, idx_hbm, out_hbm, idx_v, rows_v, sem):
    wid = lax.axis_index("s") * _NUM_CORES + lax.axis_index("c")
    base = wid * _B_PER_W
    pltpu.sync_copy(idx_hbm.at[pl.ds(base, _B_PER_W)], idx_v)
    pltpu.async_copy(table_hbm.at[idx_v], rows_v, sem).wait()
    pltpu.sync_copy(rows_v, out_hbm.at[pl.ds(base, _B_PER_W)])


@jax.jit
def _gather(embeddings, t):
    mesh = plsc.VectorSubcoreMesh(core_axis_name="c", subcore_axis_name="s")
    return pl.kernel(
        _gather_kernel,
        mesh=mesh,
        out_type=jax.ShapeDtypeStruct((_BATCH, _EMBED_DIM), jnp.float32),
        scratch_types=[
            pltpu.VMEM((_B_PER_W,), jnp.int32),
            pltpu.VMEM((_B_PER_W, _EMBED_DIM), jnp.float32),
            pltpu.SemaphoreType.DMA,
        ],
    )(embeddings, t)


def kernel(x, t, embeddings):
    return _gather(embeddings, t)
